# baseline (speedup 1.0000x reference)
import jax
import jax.numpy as jnp
from jax import lax
from jax.experimental import pallas as pl
from jax.experimental.pallas import tpu as pltpu
from jax.experimental.pallas import tpu_sc as plsc

_VOCAB = 100000
_MAXLEN = 200
_EMBED = 64
_BATCH = 4096

_NC = 2
_NS = 16
_NW = _NC * _NS
_ET = _EMBED // 8
_BT = _BATCH // 128
_BTG = 2
_NBG = _BT // _BTG
_UNIT_B = _BTG * 128
_UNITS = _MAXLEN * _NBG
_UPW = _UNITS // _NW


def _tec_body(xt, tok, pos, out, rows0, rows1, obuf0, obuf1, idx0, idx1,
              pos_v, gsem0, gsem1, osem0, osem1):
    rows = (rows0, rows1)
    obuf = (obuf0, obuf1)
    idx = (idx0, idx1)
    gsem = (gsem0, gsem1)
    osem = (osem0, osem1)

    wid = lax.axis_index("s") * _NC + lax.axis_index("c")
    ubase = wid * _UPW

    pltpu.sync_copy(pos, pos_v)

    def split(u):
        s = u // _NBG
        return s, u - s * _NBG

    def start_gather(u, b):
        s, g = split(u)
        pltpu.sync_copy(xt.at[s, pl.ds(g * _UNIT_B, _UNIT_B)], idx[b])
        for h in range(2):
            pltpu.async_copy(
                tok.at[idx[b].at[pl.ds(h * 128, 128)]],
                rows[b].at[pl.ds(h * 128, 128)],
                gsem[b])

    def wait_gather(b):
        for h in range(2):
            pltpu.make_async_copy(
                tok.at[idx[b].at[pl.ds(h * 128, 128)]],
                rows[b].at[pl.ds(h * 128, 128)],
                gsem[b]).wait()

    def start_out(u, b):
        s, g = split(u)
        for et in range(_ET):
            pltpu.async_copy(obuf[b].at[et],
                             out.at[s, et, pl.ds(g * _BTG, _BTG)],
                             osem[b])

    def wait_out(b):
        for et in range(_ET):
            pltpu.make_async_copy(obuf[b].at[et],
                                  out.at[0, et, pl.ds(0, _BTG)],
                                  osem[b]).wait()

    start_gather(ubase, 0)

    def outer(k, carry):
        for b in range(2):
            i = 2 * k + b
            u = ubase + i
            nb = 1 - b

            wait_gather(b)

            @pl.when(i + 1 < _UPW)
            def _prefetch():
                @pl.when(i >= 1)
                def _():
                    wait_out(nb)
                start_gather(u + 1, nb)

            rb = rows[b]
            ob = obuf[b]
            s, _g = split(u)

            i_s = lax.broadcast(s, (16,))

            def do_et(et, c2):
                for btl in range(_BTG):
                    for ei in range(8):
                        e = et * 8 + ei
                        ie = lax.broadcast(e, (16,))
                        p = plsc.load_gather(pos_v, [i_s, ie])
                        for j in range(8):
                            ib = (jnp.arange(16, dtype=jnp.int32)
                                  + (btl * 128 + j * 16))
                            v = plsc.load_gather(rb, [ib, ie])
                            ob[et, btl, ei, pl.ds(j * 16, 16)] = v + p
                return c2

            lax.fori_loop(0, _ET, do_et, 0)

            start_out(u, b)
        return carry

    lax.fori_loop(0, _UPW // 2, outer, 0)
    wait_out(0)
    wait_out(1)


def kernel(x, token_table, pos_table):
    xt = jnp.swapaxes(x.astype(jnp.int32), 0, 1)
    mesh = plsc.VectorSubcoreMesh(core_axis_name="c", subcore_axis_name="s")
    run = pl.kernel(
        _tec_body,
        out_type=jax.ShapeDtypeStruct((_MAXLEN, _ET, _BT, 8, 128),
                                      jnp.float32),
        mesh=mesh,
        compiler_params=pltpu.CompilerParams(use_tc_tiling_on_sc=False,
                                             needs_layout_passes=False),
        scratch_types=[
            pltpu.VMEM((_UNIT_B, _EMBED), jnp.float32),
            pltpu.VMEM((_UNIT_B, _EMBED), jnp.float32),
            pltpu.VMEM((_ET, _BTG, 8, 128), jnp.float32),
            pltpu.VMEM((_ET, _BTG, 8, 128), jnp.float32),
            pltpu.VMEM((_UNIT_B,), jnp.int32),
            pltpu.VMEM((_UNIT_B,), jnp.int32),
            pltpu.VMEM((_MAXLEN, _EMBED), jnp.float32),
            pltpu.SemaphoreType.DMA,
            pltpu.SemaphoreType.DMA,
            pltpu.SemaphoreType.DMA,
            pltpu.SemaphoreType.DMA,
        ],
    )
    a = run(xt, token_table, pos_table)
    return a.transpose(2, 4, 0, 1, 3).reshape(_BATCH, _MAXLEN, _EMBED)


# speedup vs baseline: 3.3289x; 3.3289x over previous
import jax
import jax.numpy as jnp
from jax import lax
from jax.experimental import pallas as pl
from jax.experimental.pallas import tpu as pltpu
from jax.experimental.pallas import tpu_sc as plsc

_VOCAB = 100000
_MAXLEN = 200
_EMBED = 64
_BATCH = 4096

_NC = 2
_NS = 16
_NW = _NC * _NS
_ET = _EMBED // 8
_BT = _BATCH // 128
_BTG = 2
_NBG = _BT // _BTG
_UNIT_B = _BTG * 128
_UNITS = _MAXLEN * _NBG
_UPW = _UNITS // _NW
_PAIR_B = 2 * _UNIT_B
_OUT_FLAT = _MAXLEN * _ET * _BT * 8 * 128


def _tec_body(xt, tok, pos, out, rows0, rows1, obuf0, obuf1, idx0, idx1,
              pos_v, gsem0, gsem1, osem0, osem1, isem0, isem1):
    rows = (rows0, rows1)
    obuf = (obuf0, obuf1)
    idx = (idx0, idx1)
    gsem = (gsem0, gsem1)
    osem = (osem0, osem1)
    isem = (isem0, isem1)

    wid = lax.axis_index("s") * _NC + lax.axis_index("c")
    ubase = wid * _UPW
    jbase = ubase // 2

    pltpu.sync_copy(pos, pos_v)

    iota = jnp.arange(16, dtype=jnp.int32)

    def split(u):
        s = u // _NBG
        return s, u - s * _NBG

    def _idx_copy(j, pb):
        s, g0 = split(2 * j)
        return pltpu.make_async_copy(
            xt.at[s, pl.ds(g0 * _UNIT_B, _PAIR_B)], idx[pb], isem[pb])

    def start_idx(j, pb):
        _idx_copy(j, pb).start()

    def start_gather(u, rp, pb, first_of_pair):
        if first_of_pair:
            _idx_copy(u // 2, pb).wait()
        h0 = 2 * (u % 2)
        for h in range(2):
            pltpu.async_copy(
                tok.at[idx[pb].at[pl.ds((h0 + h) * 128, 128)]],
                rows[rp].at[pl.ds(h * 128, 128)],
                gsem[rp])

    def wait_gather(rp, pb, h0):
        for h in range(2):
            pltpu.make_async_copy(
                tok.at[idx[pb].at[pl.ds((h0 + h) * 128, 128)]],
                rows[rp].at[pl.ds(h * 128, 128)],
                gsem[rp]).wait()

    def start_out(j, ob):
        s, g0 = split(2 * j)
        off = s * (_ET * _BT * 1024) + g0 * (_BTG * 1024)
        for et in range(_ET):
            pltpu.async_copy(obuf[ob].at[et],
                             out.at[pl.ds(off + et * _BT * 1024, 4096)],
                             osem[ob])

    def wait_out(ob):
        for et in range(_ET):
            pltpu.make_async_copy(obuf[ob].at[et],
                                  out.at[pl.ds(et * 4096, 4096)],
                                  osem[ob]).wait()

    start_idx(jbase, 0)
    start_idx(jbase + 1, 1)
    start_gather(ubase, 0, 0, True)

    def outer(jj, carry):
        for bj in range(2):
            j = 2 * jj + bj
            nbj = 1 - bj
            for p in range(2):
                i = 4 * jj + 2 * bj + p
                u = ubase + i

                wait_gather(p, bj, 2 * p)

                @pl.when(i + 1 < _UPW)
                def _prefetch():
                    start_gather(u + 1, 1 - p,
                                 bj if p == 0 else nbj, p == 1)

                if p == 0:
                    @pl.when(i >= 4)
                    def _wait_prev_out():
                        wait_out(bj)
                else:
                    @pl.when(j < _UPW // 2 - 2)
                    def _next_idx():
                        start_idx(jbase + j + 2, bj)

                rb = rows[p]
                ob = obuf[bj]
                s, _g = split(u)
                i_s = lax.broadcast(s, (16,))

                for c in range(_EMBED // 16):
                    @plsc.parallel_loop(0, 16, unroll=2)
                    def _diag(d):
                        perm = (iota + d) & 15
                        ie = perm + (16 * c)
                        pr = plsc.load_gather(pos_v, [i_s, ie])
                        orow = (perm >> 3) + (2 * c)
                        ocol = (perm & 7) * 128 + iota
                        for t in range(16):
                            ib = iota + (16 * t)
                            v = plsc.load_gather(rb, [ib, ie])
                            w = v + pr
                            dst = ocol + (p * 2048 + 1024 * (t // 8)
                                          + 16 * (t % 8))
                            plsc.store_scatter(ob, [orow, dst], w)

                if p == 1:
                    start_out(jbase + j, bj)
        return carry

    lax.fori_loop(0, _UPW // 4, outer, 0)
    wait_out(0)
    wait_out(1)


def kernel(x, token_table, pos_table):
    xt = jnp.swapaxes(x.astype(jnp.int32), 0, 1)
    mesh = plsc.VectorSubcoreMesh(core_axis_name="c", subcore_axis_name="s")
    run = pl.kernel(
        _tec_body,
        out_type=jax.ShapeDtypeStruct((_OUT_FLAT,), jnp.float32),
        mesh=mesh,
        compiler_params=pltpu.CompilerParams(use_tc_tiling_on_sc=False,
                                             needs_layout_passes=False),
        scratch_types=[
            pltpu.VMEM((_UNIT_B, _EMBED), jnp.float32),
            pltpu.VMEM((_UNIT_B, _EMBED), jnp.float32),
            pltpu.VMEM((_ET, 4096), jnp.float32),
            pltpu.VMEM((_ET, 4096), jnp.float32),
            pltpu.VMEM((_PAIR_B,), jnp.int32),
            pltpu.VMEM((_PAIR_B,), jnp.int32),
            pltpu.VMEM((_MAXLEN, _EMBED), jnp.float32),
            pltpu.SemaphoreType.DMA,
            pltpu.SemaphoreType.DMA,
            pltpu.SemaphoreType.DMA,
            pltpu.SemaphoreType.DMA,
            pltpu.SemaphoreType.DMA,
            pltpu.SemaphoreType.DMA,
        ],
    )
    a = run(xt, token_table, pos_table)
    return (a.reshape(_MAXLEN, _ET, _BT, 8, 128)
            .transpose(2, 4, 0, 1, 3)
            .reshape(_BATCH, _MAXLEN, _EMBED))


# speedup vs baseline: 4.2419x; 1.2743x over previous
import jax
import jax.numpy as jnp
from jax import lax
from jax.experimental import pallas as pl
from jax.experimental.pallas import tpu as pltpu
from jax.experimental.pallas import tpu_sc as plsc

_VOCAB = 100000
_MAXLEN = 200
_EMBED = 64
_BATCH = 4096

_NC = 2
_NS = 16
_NW = _NC * _NS
_ET = _EMBED // 8
_BT = _BATCH // 128
_BTG = 2
_NBG = _BT // _BTG
_UNIT_B = _BTG * 128
_UNITS = _MAXLEN * _NBG
_UPW = _UNITS // _NW
_OBUF = _ET * _BTG * 8 * 128
_OUT_FLAT = _MAXLEN * _ET * _BT * 8 * 128


def _tec_body(xt, tok, pos, out, rows0, rows1, obuf0, obuf1, idx0, idx1,
              pos_v, gsem0, gsem1, osem0, osem1, isem0, isem1):
    rows = (rows0, rows1)
    obuf = (obuf0, obuf1)
    idx = (idx0, idx1)
    gsem = (gsem0, gsem1)
    osem = (osem0, osem1)
    isem = (isem0, isem1)

    wid = lax.axis_index("s") * _NC + lax.axis_index("c")
    ubase = wid * _UPW

    pltpu.sync_copy(pos, pos_v)

    iota = jnp.arange(16, dtype=jnp.int32)

    def split(u):
        s = u // _NBG
        return s, u - s * _NBG

    def start_idx(u, b):
        s, g = split(u)
        pltpu.async_copy(xt.at[s, pl.ds(g * _UNIT_B, _UNIT_B)], idx[b],
                         isem[b])

    def start_gather(u, b):
        s, g = split(u)
        pltpu.make_async_copy(xt.at[s, pl.ds(g * _UNIT_B, _UNIT_B)], idx[b],
                              isem[b]).wait()
        for h in range(2):
            pltpu.async_copy(
                tok.at[idx[b].at[pl.ds(h * 128, 128)]],
                rows[b].at[pl.ds(h * 128, 128)],
                gsem[b])

    def wait_gather(b):
        for h in range(2):
            pltpu.make_async_copy(
                tok.at[idx[b].at[pl.ds(h * 128, 128)]],
                rows[b].at[pl.ds(h * 128, 128)],
                gsem[b]).wait()

    def start_out(u, b):
        s, g = split(u)
        off = s * (_ET * _BT * 1024) + g * (_BTG * 1024)
        for et in range(_ET):
            pltpu.async_copy(obuf[b].at[pl.ds(et * _BTG * 1024, _BTG * 1024)],
                             out.at[pl.ds(off + et * _BT * 1024,
                                          _BTG * 1024)],
                             osem[b])

    def wait_out(b):
        for et in range(_ET):
            pltpu.make_async_copy(
                obuf[b].at[pl.ds(et * _BTG * 1024, _BTG * 1024)],
                out.at[pl.ds(et * _BTG * 1024, _BTG * 1024)],
                osem[b]).wait()

    start_idx(ubase, 0)
    start_idx(ubase + 1, 1)
    start_gather(ubase, 0)

    def outer(k, carry):
        for b in range(2):
            i = 2 * k + b
            u = ubase + i
            nb = 1 - b

            wait_gather(b)

            @pl.when(i + 1 < _UPW)
            def _prefetch():
                @pl.when(i >= 1)
                def _():
                    wait_out(nb)
                start_gather(u + 1, nb)

            @pl.when(i + 2 < _UPW)
            def _prefetch_idx():
                start_idx(u + 2, b)

            rb = rows[b]
            ob = obuf[b]
            s, _g = split(u)
            i_s = lax.broadcast(s, (16,))

            for c in range(_EMBED // 16):
                @plsc.parallel_loop(0, 16, unroll=2)
                def _diag(d):
                    perm = (iota + d) & 15
                    ie = perm + (16 * c)
                    pr = plsc.load_gather(pos_v, [i_s, ie])
                    vs = ((perm >> 3) * 2048 + (perm & 7) * 128 + iota
                          + (4096 * c))
                    for t in range(16):
                        ib = iota + (16 * t)
                        v = plsc.load_gather(rb, [ib, ie])
                        w = v + pr
                        dst = vs + (1024 * (t // 8) + 16 * (t % 8))
                        plsc.store_scatter(ob, [dst], w)

            start_out(u, b)
        return carry

    lax.fori_loop(0, _UPW // 2, outer, 0)
    wait_out(0)
    wait_out(1)


def kernel(x, token_table, pos_table):
    xt = jnp.swapaxes(x.astype(jnp.int32), 0, 1)
    mesh = plsc.VectorSubcoreMesh(core_axis_name="c", subcore_axis_name="s")
    run = pl.kernel(
        _tec_body,
        out_type=jax.ShapeDtypeStruct((_OUT_FLAT,), jnp.float32),
        mesh=mesh,
        compiler_params=pltpu.CompilerParams(use_tc_tiling_on_sc=False,
                                             needs_layout_passes=False),
        scratch_types=[
            pltpu.VMEM((_UNIT_B, _EMBED), jnp.float32),
            pltpu.VMEM((_UNIT_B, _EMBED), jnp.float32),
            pltpu.VMEM((_OBUF,), jnp.float32),
            pltpu.VMEM((_OBUF,), jnp.float32),
            pltpu.VMEM((_UNIT_B,), jnp.int32),
            pltpu.VMEM((_UNIT_B,), jnp.int32),
            pltpu.VMEM((_MAXLEN, _EMBED), jnp.float32),
            pltpu.SemaphoreType.DMA,
            pltpu.SemaphoreType.DMA,
            pltpu.SemaphoreType.DMA,
            pltpu.SemaphoreType.DMA,
            pltpu.SemaphoreType.DMA,
            pltpu.SemaphoreType.DMA,
        ],
    )
    a = run(xt, token_table, pos_table)
    return (a.reshape(_MAXLEN, _ET, _BT, 8, 128)
            .transpose(2, 4, 0, 1, 3)
            .reshape(_BATCH, _MAXLEN, _EMBED))


# speedup vs baseline: 4.6666x; 1.1001x over previous
import jax
import jax.numpy as jnp
from jax import lax
from jax.experimental import pallas as pl
from jax.experimental.pallas import tpu as pltpu
from jax.experimental.pallas import tpu_sc as plsc

_VOCAB = 100000
_MAXLEN = 200
_EMBED = 64
_BATCH = 4096

_NC = 2
_NS = 16
_NW = _NC * _NS
_ET = _EMBED // 8
_BT = _BATCH // 128
_BTG = 2
_NBG = _BT // _BTG
_UNIT_B = _BTG * 128
_UNITS = _MAXLEN * _NBG
_UPW = _UNITS // _NW
_OBUF = _ET * _BTG * 8 * 128
_OUT_FLAT = _MAXLEN * _ET * _BT * 8 * 128


def _tec_body(xt, tok, pos, out, rows0, rows1, obuf0, obuf1, idx0, idx1,
              pos_v, gsem0, gsem1, osem0, osem1, isem0, isem1):
    rows = (rows0, rows1)
    obuf = (obuf0, obuf1)
    idx = (idx0, idx1)
    gsem = (gsem0, gsem1)
    osem = (osem0, osem1)
    isem = (isem0, isem1)

    wid = lax.axis_index("s") * _NC + lax.axis_index("c")
    ubase = wid * _UPW

    pltpu.sync_copy(pos, pos_v)

    iota = jnp.arange(16, dtype=jnp.int32)

    def split(u):
        s = u // _NBG
        return s, u - s * _NBG

    def start_idx(u, b):
        s, g = split(u)
        pltpu.async_copy(xt.at[s, pl.ds(g * _UNIT_B, _UNIT_B)], idx[b],
                         isem[b])

    def start_gather(u, b):
        s, g = split(u)
        pltpu.make_async_copy(xt.at[s, pl.ds(g * _UNIT_B, _UNIT_B)], idx[b],
                              isem[b]).wait()
        for h in range(2):
            pltpu.async_copy(
                tok.at[idx[b].at[pl.ds(h * 128, 128)]],
                rows[b].at[pl.ds(h * 128, 128)],
                gsem[b])

    def wait_gather(b):
        for h in range(2):
            pltpu.make_async_copy(
                tok.at[idx[b].at[pl.ds(h * 128, 128)]],
                rows[b].at[pl.ds(h * 128, 128)],
                gsem[b]).wait()

    def start_out(u, b):
        s, g = split(u)
        off = s * (_ET * _BT * 1024) + g * (_BTG * 1024)
        for et in range(_ET):
            pltpu.async_copy(obuf[b].at[pl.ds(et * _BTG * 1024, _BTG * 1024)],
                             out.at[pl.ds(off + et * _BT * 1024,
                                          _BTG * 1024)],
                             osem[b])

    def wait_out(b):
        for et in range(_ET):
            pltpu.make_async_copy(
                obuf[b].at[pl.ds(et * _BTG * 1024, _BTG * 1024)],
                out.at[pl.ds(et * _BTG * 1024, _BTG * 1024)],
                osem[b]).wait()

    start_idx(ubase, 0)
    start_idx(ubase + 1, 1)
    start_gather(ubase, 0)

    def outer(k, carry):
        for b in range(2):
            i = 2 * k + b
            u = ubase + i
            nb = 1 - b

            wait_gather(b)

            @pl.when(i + 1 < _UPW)
            def _prefetch():
                start_gather(u + 1, nb)

            @pl.when(i + 2 < _UPW)
            def _prefetch_idx():
                start_idx(u + 2, b)

            @pl.when(i >= 2)
            def _wait_prev_out():
                wait_out(b)

            rb = rows[b]
            ob = obuf[b]
            s, _g = split(u)
            i_s = lax.broadcast(s, (16,))

            for c in range(_EMBED // 16):
                @plsc.parallel_loop(0, 16, unroll=2)
                def _diag(d):
                    perm = (iota + d) & 15
                    ie = perm + (16 * c)
                    pr = plsc.load_gather(pos_v, [i_s, ie])
                    vs = ((perm >> 3) * 2048 + (perm & 7) * 128 + iota
                          + (4096 * c))
                    for t in range(16):
                        ib = iota + (16 * t)
                        v = plsc.load_gather(rb, [ib, ie])
                        w = v + pr
                        dst = vs + (1024 * (t // 8) + 16 * (t % 8))
                        plsc.store_scatter(ob, [dst], w)

            start_out(u, b)
        return carry

    lax.fori_loop(0, _UPW // 2, outer, 0)
    wait_out(0)
    wait_out(1)


def kernel(x, token_table, pos_table):
    xt = jnp.swapaxes(x.astype(jnp.int32), 0, 1)
    mesh = plsc.VectorSubcoreMesh(core_axis_name="c", subcore_axis_name="s")
    run = pl.kernel(
        _tec_body,
        out_type=jax.ShapeDtypeStruct((_OUT_FLAT,), jnp.float32),
        mesh=mesh,
        compiler_params=pltpu.CompilerParams(use_tc_tiling_on_sc=False,
                                             needs_layout_passes=False),
        scratch_types=[
            pltpu.VMEM((_UNIT_B, _EMBED), jnp.float32),
            pltpu.VMEM((_UNIT_B, _EMBED), jnp.float32),
            pltpu.VMEM((_OBUF,), jnp.float32),
            pltpu.VMEM((_OBUF,), jnp.float32),
            pltpu.VMEM((_UNIT_B,), jnp.int32),
            pltpu.VMEM((_UNIT_B,), jnp.int32),
            pltpu.VMEM((_MAXLEN, _EMBED), jnp.float32),
            pltpu.SemaphoreType.DMA,
            pltpu.SemaphoreType.DMA,
            pltpu.SemaphoreType.DMA,
            pltpu.SemaphoreType.DMA,
            pltpu.SemaphoreType.DMA,
            pltpu.SemaphoreType.DMA,
        ],
    )
    a = run(xt, token_table, pos_table)
    return (a.reshape(_MAXLEN, _ET, _BT, 8, 128)
            .transpose(2, 4, 0, 1, 3)
            .reshape(_BATCH, _MAXLEN, _EMBED))
